# baseline (speedup 1.0000x reference)
import functools

import jax
import jax.numpy as jnp
from jax import lax
from jax.experimental import pallas as pl
from jax.experimental.pallas import tpu as pltpu
from jax.experimental.pallas import tpu_sc as plsc

_NUM_CORES = 2
_NUM_SUBCORES = 16
_NSHIFT = 16
_PLANE = 4096
_BPW = 64
_WIN = 16 * (_BPW - 1) + 2048
_H_SC = 8
_TSHIFT = 128
_TBLK = 128


@functools.partial(jax.jit, static_argnums=(1, 2))
def _expand_bias_sc(rev16, HS, S):
    mesh = plsc.VectorSubcoreMesh(core_axis_name="c", subcore_axis_name="s")

    @functools.partial(
        pl.kernel,
        out_type=jax.ShapeDtypeStruct((HS, S, S), jnp.float32),
        mesh=mesh,
        scratch_types=[
            pltpu.VMEM((HS, _WIN), jnp.float32),
            pltpu.SemaphoreType.DMA,
            pltpu.SemaphoreType.DMA,
        ],
        compiler_params=pltpu.CompilerParams(use_tc_tiling_on_sc=False),
    )
    def body(rev_hbm, out_hbm, buf, gsem, ssem):
        wid = lax.axis_index("s") * _NUM_CORES + lax.axis_index("c")
        rho = wid % _NSHIFT
        half = wid // _NSHIFT
        b0 = half * _BPW
        s = (_NSHIFT - 1) - rho
        qmin = (S // _NSHIFT) - b0 - _BPW

        pltpu.make_async_copy(
            rev_hbm.at[s, :, pl.ds(qmin * _NSHIFT, _WIN)], buf, gsem
        ).start()
        pltpu.make_async_copy(
            rev_hbm.at[0, :, pl.ds(0, _WIN)], buf, gsem
        ).wait()

        def issue(t, carry):
            i = rho + _NSHIFT * (b0 + t)
            pltpu.make_async_copy(
                buf.at[:, pl.ds(_NSHIFT * (_BPW - 1 - t), S)],
                out_hbm.at[:, i, :],
                ssem,
            ).start()
            return carry

        lax.fori_loop(0, _BPW, issue, 0)
        pltpu.make_async_copy(
            out_hbm.at[:, pl.ds(0, _BPW), :],
            out_hbm.at[:, pl.ds(0, _BPW), :],
            ssem,
        ).wait()

    return body(rev16)


@functools.partial(jax.jit, static_argnums=(1, 2))
def _expand_bias_tc(rev128c, HT, S):
    ABLK = S // _TBLK

    def body(in_ref, out_ref):
        a = pl.program_id(1)
        q = (ABLK - 1) - a
        off = pl.multiple_of(q * _TBLK, 128)
        out_ref[0] = in_ref[0, :, pl.ds(off, S)]

    return pl.pallas_call(
        body,
        grid=(HT, ABLK),
        in_specs=[pl.BlockSpec((1, _TSHIFT, _PLANE), lambda h, a: (h, 0, 0))],
        out_specs=pl.BlockSpec((1, _TBLK, S), lambda h, a: (h, a, 0)),
        out_shape=jax.ShapeDtypeStruct((HT, S, S), jnp.float32),
        compiler_params=pltpu.CompilerParams(
            dimension_semantics=("parallel", "arbitrary"),
        ),
    )(rev128c)


def kernel(seq_len, table):
    del seq_len
    R, H = table.shape
    S = (R + 1) // 2
    rev = table[::-1, :].T
    rev_pad = jnp.pad(rev, ((0, 0), (0, _PLANE + _TSHIFT - 1 - rev.shape[1])))
    rev16 = jnp.stack(
        [rev_pad[:_H_SC, s:s + _PLANE] for s in range(_NSHIFT)]
    )
    rev128c = jnp.stack(
        [rev_pad[_H_SC:, s:s + _PLANE] for s in reversed(range(_TSHIFT))],
        axis=1,
    )
    sc_rows = _expand_bias_sc(rev16, _H_SC, S)
    tc_rows = _expand_bias_tc(rev128c, H - _H_SC, S)
    rows = jnp.concatenate([sc_rows, tc_rows], axis=0)
    return rows.reshape(1, H, S, S)


# speedup vs baseline: 1.4704x; 1.4704x over previous
import functools

import jax
import jax.numpy as jnp
from jax import lax
from jax.experimental import pallas as pl
from jax.experimental.pallas import tpu as pltpu
from jax.experimental.pallas import tpu_sc as plsc

_NUM_CORES = 2
_NUM_SUBCORES = 16
_NSHIFT = 16
_PLANE = 4096
_BPW = 64
_WIN = 16 * (_BPW - 1) + 2048


@functools.partial(jax.jit, static_argnums=(1, 2))
def _expand_bias(rev16, H, S):
    mesh = plsc.VectorSubcoreMesh(core_axis_name="c", subcore_axis_name="s")

    @functools.partial(
        pl.kernel,
        out_type=jax.ShapeDtypeStruct((H, S, S), jnp.float32),
        mesh=mesh,
        scratch_types=[
            pltpu.VMEM((H, _WIN), jnp.float32),
            pltpu.SemaphoreType.DMA,
            pltpu.SemaphoreType.DMA,
        ],
        compiler_params=pltpu.CompilerParams(use_tc_tiling_on_sc=False),
    )
    def body(rev_hbm, out_hbm, buf, gsem, ssem):
        wid = lax.axis_index("s") * _NUM_CORES + lax.axis_index("c")
        rho = wid % _NSHIFT
        half = wid // _NSHIFT
        b0 = half * _BPW
        s = (_NSHIFT - 1) - rho
        qmin = (S // _NSHIFT) - b0 - _BPW

        pltpu.make_async_copy(
            rev_hbm.at[s, :, pl.ds(qmin * _NSHIFT, _WIN)], buf, gsem
        ).start()
        pltpu.make_async_copy(
            rev_hbm.at[0, :, pl.ds(0, _WIN)], buf, gsem
        ).wait()

        def issue(t, carry):
            i = rho + _NSHIFT * (b0 + t)
            pltpu.make_async_copy(
                buf.at[:, pl.ds(_NSHIFT * (_BPW - 1 - t), S)],
                out_hbm.at[:, i, :],
                ssem,
            ).start()
            return carry

        lax.fori_loop(0, _BPW, issue, 0)
        pltpu.make_async_copy(
            out_hbm.at[:, pl.ds(0, _BPW), :],
            out_hbm.at[:, pl.ds(0, _BPW), :],
            ssem,
        ).wait()

    return body(rev16)


def kernel(seq_len, table):
    del seq_len
    R, H = table.shape
    S = (R + 1) // 2
    rev = table[::-1, :].T
    rev_pad = jnp.pad(rev, ((0, 0), (0, _PLANE + _NSHIFT - 1 - rev.shape[1])))
    rev16 = jnp.stack([rev_pad[:, s:s + _PLANE] for s in range(_NSHIFT)])
    rows = _expand_bias(rev16, H, S)
    return rows.reshape(1, H, S, S)
